# baseline (speedup 1.0000x reference)
import functools

import jax
import jax.numpy as jnp
from jax import lax
from jax.experimental import pallas as pl
from jax.experimental.pallas import tpu as pltpu
from jax.experimental.pallas import tpu_sc as plsc


def _sc_gather(table, idx):
    V, D = table.shape
    B = idx.shape[0]
    info = plsc.get_sparse_core_info()
    num_workers = info.num_cores * info.num_subcores
    b_per_w = B // num_workers
    mesh = plsc.VectorSubcoreMesh(core_axis_name="c", subcore_axis_name="s")

    @functools.partial(
        pl.kernel,
        mesh=mesh,
        out_type=jax.ShapeDtypeStruct((B, D), jnp.float32),
        scratch_types=[
            pltpu.VMEM((b_per_w,), jnp.int32),
            pltpu.VMEM((b_per_w, D), jnp.float32),
            pltpu.SemaphoreType.DMA,
        ],
        compiler_params=pltpu.CompilerParams(
            use_tc_tiling_on_sc=True, needs_layout_passes=False
        ),
    )
    def gather_kernel(table_hbm, idx_hbm, out_hbm, idx_v, rows_v, sem):
        wid = lax.axis_index("s") * info.num_cores + lax.axis_index("c")
        base = wid * b_per_w
        pltpu.sync_copy(idx_hbm.at[pl.ds(base, b_per_w)], idx_v)

        def issue_group(g, _):
            v = idx_v[pl.ds(g * 16, 16)]
            for k in range(16):
                pltpu.async_copy(
                    table_hbm.at[v[k]],
                    rows_v.at[g * 16 + k],
                    sem,
                )
            return _

        lax.fori_loop(0, b_per_w // 16, issue_group, None)

        def drain(j, _):
            pltpu.make_async_copy(
                table_hbm.at[0], rows_v.at[0], sem
            ).wait()
            return _

        lax.fori_loop(0, b_per_w, drain, None)
        pltpu.sync_copy(rows_v, out_hbm.at[pl.ds(base, b_per_w)])

    return gather_kernel(table, idx)


def _tc_mlp(x, W1, b1, W2, b2):
    B, D = x.shape
    H = W1.shape[1]
    BLK = 2048

    def body(x_ref, w1_ref, b1_ref, w2_ref, b2_ref, o_ref):
        xb = x_ref[...]
        h = jnp.maximum(
            jnp.dot(xb, w1_ref[...], preferred_element_type=jnp.float32)
            + b1_ref[...],
            0.0,
        )
        o_ref[...] = jnp.maximum(
            xb
            + jnp.dot(h, w2_ref[...], preferred_element_type=jnp.float32)
            + b2_ref[...],
            0.0,
        )

    return pl.pallas_call(
        body,
        grid=(B // BLK,),
        in_specs=[
            pl.BlockSpec((BLK, D), lambda i: (i, 0)),
            pl.BlockSpec((D, H), lambda i: (0, 0)),
            pl.BlockSpec((1, H), lambda i: (0, 0)),
            pl.BlockSpec((H, D), lambda i: (0, 0)),
            pl.BlockSpec((1, D), lambda i: (0, 0)),
        ],
        out_specs=pl.BlockSpec((BLK, D), lambda i: (i, 0)),
        out_shape=jax.ShapeDtypeStruct((B, D), jnp.float32),
    )(x, W1, b1.reshape(1, H), W2, b2.reshape(1, D))


def kernel(a, table, W1, b1, W2, b2):
    x = _sc_gather(table, a.astype(jnp.int32))
    return _tc_mlp(x, W1, b1, W2, b2)


# speedup vs baseline: 1.5059x; 1.5059x over previous
import functools

import jax
import jax.numpy as jnp
from jax import lax
from jax.experimental import pallas as pl
from jax.experimental.pallas import tpu as pltpu
from jax.experimental.pallas import tpu_sc as plsc


def _sc_gather(table, idx):
    V, D = table.shape
    B = idx.shape[0]
    info = plsc.get_sparse_core_info()
    num_workers = info.num_cores * info.num_subcores
    b_per_w = B // num_workers
    table3 = table.reshape(V // 8, 8, D)
    mesh = plsc.VectorSubcoreMesh(core_axis_name="c", subcore_axis_name="s")

    @functools.partial(
        pl.kernel,
        mesh=mesh,
        out_type=jax.ShapeDtypeStruct((B, D), jnp.float32),
        scratch_types=[
            pltpu.VMEM((b_per_w,), jnp.int32),
            pltpu.VMEM((b_per_w, D), jnp.float32),
            pltpu.SemaphoreType.DMA,
        ],
        compiler_params=pltpu.CompilerParams(
            use_tc_tiling_on_sc=True, needs_layout_passes=False
        ),
    )
    def gather_kernel(table_hbm, idx_hbm, out_hbm, idx_v, rows_v, sem):
        wid = lax.axis_index("s") * info.num_cores + lax.axis_index("c")
        base = wid * b_per_w
        pltpu.sync_copy(idx_hbm.at[pl.ds(base, b_per_w)], idx_v)

        def issue_group(g, _):
            v = idx_v[pl.ds(g * 16, 16)]
            t16 = v >> 3
            s16 = v & 7
            for k in range(16):
                pltpu.async_copy(
                    table_hbm.at[t16[k], s16[k]],
                    rows_v.at[g * 16 + k],
                    sem,
                )
            return _

        lax.fori_loop(0, b_per_w // 16, issue_group, None)

        def drain(j, _):
            pltpu.make_async_copy(
                table_hbm.at[0, 0], rows_v.at[0], sem
            ).wait()
            return _

        lax.fori_loop(0, b_per_w, drain, None)
        pltpu.sync_copy(rows_v, out_hbm.at[pl.ds(base, b_per_w)])

    return gather_kernel(table3, idx)


def _tc_mlp(x, W1, b1, W2T, b2):
    B, D = x.shape
    H = W1.shape[1]
    BLK = 2048
    dn = (((1,), (1,)), ((), ()))

    def body(x_ref, w1_ref, b1_ref, w2t_ref, b2_ref, o_ref):
        xb = x_ref[...]
        h = jnp.maximum(
            jnp.dot(xb, w1_ref[...], preferred_element_type=jnp.float32)
            + b1_ref[...],
            0.0,
        )
        y = jnp.maximum(
            xb
            + lax.dot_general(
                h, w2t_ref[...], dn, preferred_element_type=jnp.float32
            )
            + b2_ref[...],
            0.0,
        )
        o_ref[...] = y.T

    return pl.pallas_call(
        body,
        grid=(B // BLK,),
        in_specs=[
            pl.BlockSpec((BLK, D), lambda i: (i, 0)),
            pl.BlockSpec((D, H), lambda i: (0, 0)),
            pl.BlockSpec((1, H), lambda i: (0, 0)),
            pl.BlockSpec((D, H), lambda i: (0, 0)),
            pl.BlockSpec((1, D), lambda i: (0, 0)),
        ],
        out_specs=pl.BlockSpec((D, BLK), lambda i: (0, i)),
        out_shape=jax.ShapeDtypeStruct((D, B), jnp.float32),
    )(x, W1, b1.reshape(1, H), W2T, b2.reshape(1, D))


def kernel(a, table, W1, b1, W2, b2):
    x = _sc_gather(table, a.astype(jnp.int32))
    outT = _tc_mlp(x, W1, b1, W2.T, b2)
    return outT.T
